# baseline (speedup 1.0000x reference)
import functools

import jax
import jax.numpy as jnp
from jax import lax
from jax.experimental import pallas as pl
from jax.experimental.pallas import tpu as pltpu
from jax.experimental.pallas import tpu_sc as plsc

N = 10000
E = 320000
B = 20
FEAT = 128
H = 128

NC = 2
NS = 16
CHUNK = 256
CHUNKS = 79
EDGES_PER_S = CHUNKS * CHUNK
E_PAD = EDGES_PER_S * NS
N_PAD = 10112
ROWS_PER_S = N_PAD // NS


def _sc_segsum(table0, table1, src_pad, dst_pad, zrows):
    mesh = plsc.VectorSubcoreMesh(core_axis_name="c", subcore_axis_name="s")

    @functools.partial(
        pl.kernel,
        mesh=mesh,
        out_type=[
            jax.ShapeDtypeStruct((N_PAD, 128), jnp.float32),
            jax.ShapeDtypeStruct((N_PAD, 128), jnp.float32),
        ],
        scratch_types=[
            pltpu.VMEM((CHUNK,), jnp.int32),
            pltpu.VMEM((CHUNK,), jnp.int32),
            pltpu.VMEM((CHUNK, 128), jnp.float32),
            pltpu.VMEM_SHARED((N_PAD, 128), jnp.float32),
            pltpu.SemaphoreType.DMA,
        ],
    )
    def k(t0_hbm, t1_hbm, src_hbm, dst_hbm, z_hbm, out0_hbm, out1_hbm,
          sidx, didx, gbuf, acc, gsem):
        cid = lax.axis_index("c")
        sid = lax.axis_index("s")
        ebase = sid * EDGES_PER_S
        rbase = sid * ROWS_PER_S

        pltpu.sync_copy(z_hbm.at[pl.ds(rbase, ROWS_PER_S)],
                        acc.at[pl.ds(rbase, ROWS_PER_S)])
        plsc.subcore_barrier()

        def body_for(tab):
            def body(g, carry):
                off = ebase + g * CHUNK
                pltpu.sync_copy(src_hbm.at[pl.ds(off, CHUNK)], sidx)
                pltpu.sync_copy(dst_hbm.at[pl.ds(off, CHUNK)], didx)
                pltpu.async_copy(tab.at[sidx], gbuf, gsem).wait()
                pltpu.sync_copy(gbuf, acc.at[didx], add=True)
                return carry
            return body

        @pl.when(cid == 0)
        def _():
            lax.fori_loop(0, CHUNKS, body_for(t0_hbm), 0)

        @pl.when(cid == 1)
        def _():
            lax.fori_loop(0, CHUNKS, body_for(t1_hbm), 0)

        plsc.subcore_barrier()

        @pl.when(cid == 0)
        def _():
            pltpu.sync_copy(acc.at[pl.ds(rbase, ROWS_PER_S)],
                            out0_hbm.at[pl.ds(rbase, ROWS_PER_S)])

        @pl.when(cid == 1)
        def _():
            pltpu.sync_copy(acc.at[pl.ds(rbase, ROWS_PER_S)],
                            out1_hbm.at[pl.ds(rbase, ROWS_PER_S)])

    return k(table0, table1, src_pad, dst_pad, zrows)


RB = 1000
NRB = N // RB


def _ln_in(h, g, b):
    mu = jnp.mean(h, axis=-1, keepdims=True)
    var = jnp.var(h, axis=-1, keepdims=True)
    return (h - mu) / jnp.sqrt(var + 1e-5) * g + b


def _tables(nx, t):
    r = nx + 1e-7
    p = jnp.exp(r * t)
    q = r * p
    return (jnp.concatenate([p[:, :64], q[:, :64]], axis=1),
            jnp.concatenate([p[:, 64:], q[:, 64:]], axis=1))


def _prep_body(x_ref, w_ref, b_ref, t_ref, y_ref, t0_ref, t1_ref):
    y = jnp.maximum(x_ref[...] @ w_ref[...] + b_ref[...], 0.0)
    y_ref[...] = y
    t0_ref[...], t1_ref[...] = _tables(y, t_ref[0, 0])


def _tc_prep(x, wfc, bfc, t):
    row = pl.BlockSpec((RB, 128), lambda i: (i, 0))
    return pl.pallas_call(
        _prep_body,
        grid=(NRB,),
        in_specs=[
            row,
            pl.BlockSpec((128, 128), lambda i: (0, 0)),
            pl.BlockSpec((1, 128), lambda i: (0, 0)),
            pl.BlockSpec((1, 1), lambda i: (0, 0)),
        ],
        out_specs=[row, row, row],
        out_shape=[jax.ShapeDtypeStruct((N, 128), jnp.float32)] * 3,
    )(x, wfc, bfc.reshape(1, 128), t.reshape(1, 1))


def _layer_body(l, has_next, o0_ref, o1_ref, x_ref, w1_ref, b1_ref, g1_ref,
                e1_ref, w2_ref, b2_ref, gn_ref, bn_ref, t_ref, *out_refs):
    nx_ref = out_refs[0]
    o0 = o0_ref[...]
    o1 = o1_ref[...]
    xb = x_ref[...]
    S = jnp.concatenate([o0[:, :64], o1[:, :64]], axis=1)
    W = jnp.concatenate([o0[:, 64:], o1[:, 64:]], axis=1)
    o = W / (S + 1e-16) + xb
    h = o @ w1_ref[...] + b1_ref[...]
    h = jnp.maximum(_ln_in(h, g1_ref[...], e1_ref[...]), 0.0)
    h2 = h @ w2_ref[...] + b2_ref[...]
    if l == 0:
        nx = h2
    else:
        nx = xb + jnp.maximum(_ln_in(h2, gn_ref[...], bn_ref[...]), 0.0)
    nx_ref[...] = nx
    if has_next:
        t0, t1 = _tables(jnp.maximum(nx, 0.0), t_ref[0, 0])
        out_refs[1][...] = t0
        out_refs[2][...] = t1


def _tc_layer(l, out0, out1, x, p):
    has_next = l < 2
    row = pl.BlockSpec((RB, 128), lambda i: (i, 0))
    c1 = pl.BlockSpec((1, 256), lambda i: (0, 0))
    cw1 = pl.BlockSpec((128, 256), lambda i: (0, 0))
    cw2 = pl.BlockSpec((256, 128), lambda i: (0, 0))
    c2 = pl.BlockSpec((1, 128), lambda i: (0, 0))
    ct = pl.BlockSpec((1, 1), lambda i: (0, 0))
    gn = p['gn_%d' % l] if l else p['bm2_0']
    bn = p['bn_%d' % l] if l else p['bm2_0']
    tn = p['t%d' % (l + 1)] if has_next else p['t0']
    n_out = 3 if has_next else 1
    res = pl.pallas_call(
        functools.partial(_layer_body, l, has_next),
        grid=(NRB,),
        in_specs=[row, row, row, cw1, c1, c1, c1, cw2, c2, c2, c2, ct],
        out_specs=[row] * n_out,
        out_shape=[jax.ShapeDtypeStruct((N, 128), jnp.float32)] * n_out,
    )(out0[:N], out1[:N], x,
      p['Wm1_%d' % l], p['bm1_%d' % l].reshape(1, 256),
      p['g1_%d' % l].reshape(1, 256), p['be1_%d' % l].reshape(1, 256),
      p['Wm2_%d' % l], p['bm2_%d' % l].reshape(1, 128),
      gn.reshape(1, 128), bn.reshape(1, 128), tn.reshape(1, 1))
    return res if has_next else (res[0], None, None)


def _attn_body(x_ref, wphi_ref, bphi_ref, wa_ref, ba_ref, wb_ref, bb_ref,
               wc_ref, bc_ref, wr_ref, br_ref, wcls_ref, bcls_ref, out_ref):
    xb = x_ref[0]
    hp = jnp.maximum(xb @ wphi_ref[...] + bphi_ref[...], 0.0)
    a = jnp.tanh(hp @ wa_ref[...] + ba_ref[...])
    g = jax.nn.sigmoid(hp @ wb_ref[...] + bb_ref[...])
    s = ((a * g) @ wc_ref[...] + bc_ref[...])[:, :1]
    rowid = jax.lax.broadcasted_iota(jnp.int32, (512, 1), 0)
    s = jnp.where(rowid < 500, s, -1e30)
    s = s - jnp.max(s, axis=0, keepdims=True)
    e = jnp.exp(s)
    A = e / jnp.sum(e, axis=0, keepdims=True)
    h2 = jnp.sum(A * hp, axis=0, keepdims=True)
    bag = jnp.maximum(h2 @ wr_ref[...] + br_ref[...], 0.0)
    out_ref[0] = bag @ wcls_ref[...] + bcls_ref[...]


def _tc_attn(xp, p):
    D = 4 * H
    cw = pl.BlockSpec((D, D), lambda i: (0, 0))
    cb = pl.BlockSpec((1, D), lambda i: (0, 0))
    wc_pad = jnp.pad(p['Wc'], ((0, 0), (0, 127)))
    bc_pad = jnp.pad(p['bc'], ((0, 127))).reshape(1, 128)
    wcls_pad = jnp.pad(p['Wcls'], ((0, 0), (0, 126)))
    bcls_pad = jnp.pad(p['bcls'], ((0, 126))).reshape(1, 128)
    out = pl.pallas_call(
        _attn_body,
        grid=(B,),
        in_specs=[
            pl.BlockSpec((1, 512, D), lambda i: (i, 0, 0)),
            cw, cb, cw, cb, cw, cb,
            pl.BlockSpec((D, 128), lambda i: (0, 0)),
            pl.BlockSpec((1, 128), lambda i: (0, 0)),
            cw, cb,
            pl.BlockSpec((D, 128), lambda i: (0, 0)),
            pl.BlockSpec((1, 128), lambda i: (0, 0)),
        ],
        out_specs=pl.BlockSpec((1, 1, 128), lambda i: (i, 0, 0)),
        out_shape=jax.ShapeDtypeStruct((B, 1, 128), jnp.float32),
    )(xp, p['Wphi'], p['bphi'].reshape(1, D), p['Wa'], p['ba'].reshape(1, D),
      p['Wb'], p['bb'].reshape(1, D), wc_pad, bc_pad,
      p['Wrho'], p['brho'].reshape(1, D), wcls_pad, bcls_pad)
    return out.reshape(B, 128)[:, :2]


def kernel(x, edge_index, edge_latent, y, params):
    p = params
    src = edge_index[0].astype(jnp.int32)
    dst = edge_index[1].astype(jnp.int32)
    pad = E_PAD - E
    src_pad = jnp.concatenate([src, jnp.zeros((pad,), jnp.int32)])
    dst_pad = jnp.concatenate(
        [dst, N + (jnp.arange(pad, dtype=jnp.int32) % (N_PAD - N))])
    zrows = jnp.zeros((N_PAD, 128), jnp.float32)

    x, tab0, tab1 = _tc_prep(x, p['Wfc'], p['bfc'], p['t0'])
    cols = [x]
    for l in range(3):
        s0, s1 = _sc_segsum(tab0, tab1, src_pad, dst_pad, zrows)
        x, tab0, tab1 = _tc_layer(l, s0, s1, x, p)
        cols.append(x)
    x_ = jnp.concatenate(cols, axis=-1)
    xp = jnp.pad(x_.reshape(B, 500, 4 * H), ((0, 0), (0, 12), (0, 0)))
    return _tc_attn(xp, p)


# speedup vs baseline: 1.0157x; 1.0157x over previous
import functools

import jax
import jax.numpy as jnp
from jax import lax
from jax.experimental import pallas as pl
from jax.experimental.pallas import tpu as pltpu
from jax.experimental.pallas import tpu_sc as plsc

N = 10000
E = 320000
B = 20
FEAT = 128
H = 128

NC = 2
NS = 16
CHUNK = 128
CHUNKS = 157
EDGES_PER_S = CHUNKS * CHUNK
E_PAD = EDGES_PER_S * NS
N_PAD = 10112
ROWS_PER_S = N_PAD // NS


def _sc_segsum(table0, table1, src_pad, dst_pad, zrows):
    mesh = plsc.VectorSubcoreMesh(core_axis_name="c", subcore_axis_name="s")

    @functools.partial(
        pl.kernel,
        mesh=mesh,
        out_type=[
            jax.ShapeDtypeStruct((N_PAD, 128), jnp.float32),
            jax.ShapeDtypeStruct((N_PAD, 128), jnp.float32),
        ],
        scratch_types=[
            pltpu.VMEM((CHUNK,), jnp.int32),
            pltpu.VMEM((CHUNK,), jnp.int32),
            pltpu.VMEM((CHUNK, 128), jnp.float32),
            pltpu.VMEM_SHARED((N_PAD, 128), jnp.float32),
            pltpu.SemaphoreType.DMA,
        ],
    )
    def k(t0_hbm, t1_hbm, src_hbm, dst_hbm, z_hbm, out0_hbm, out1_hbm,
          sidx, didx, gbuf, acc, gsem):
        cid = lax.axis_index("c")
        sid = lax.axis_index("s")
        ebase = sid * EDGES_PER_S
        rbase = sid * ROWS_PER_S

        pltpu.sync_copy(z_hbm.at[pl.ds(rbase, ROWS_PER_S)],
                        acc.at[pl.ds(rbase, ROWS_PER_S)])
        plsc.subcore_barrier()

        def body_for(tab):
            def body(g, carry):
                off = ebase + g * CHUNK
                pltpu.sync_copy(src_hbm.at[pl.ds(off, CHUNK)], sidx)
                pltpu.sync_copy(dst_hbm.at[pl.ds(off, CHUNK)], didx)
                pltpu.async_copy(tab.at[sidx], gbuf, gsem).wait()
                pltpu.sync_copy(gbuf, acc.at[didx], add=True)
                return carry
            return body

        @pl.when(cid == 0)
        def _():
            lax.fori_loop(0, CHUNKS, body_for(t0_hbm), 0)

        @pl.when(cid == 1)
        def _():
            lax.fori_loop(0, CHUNKS, body_for(t1_hbm), 0)

        plsc.subcore_barrier()

        @pl.when(cid == 0)
        def _():
            pltpu.sync_copy(acc.at[pl.ds(rbase, ROWS_PER_S)],
                            out0_hbm.at[pl.ds(rbase, ROWS_PER_S)])

        @pl.when(cid == 1)
        def _():
            pltpu.sync_copy(acc.at[pl.ds(rbase, ROWS_PER_S)],
                            out1_hbm.at[pl.ds(rbase, ROWS_PER_S)])

    return k(table0, table1, src_pad, dst_pad, zrows)


RB = 1000
NRB = N // RB


def _ln_in(h, g, b):
    mu = jnp.mean(h, axis=-1, keepdims=True)
    var = jnp.var(h, axis=-1, keepdims=True)
    return (h - mu) / jnp.sqrt(var + 1e-5) * g + b


def _tables(nx, t):
    r = nx + 1e-7
    p = jnp.exp(r * t)
    q = r * p
    return (jnp.concatenate([p[:, :64], q[:, :64]], axis=1),
            jnp.concatenate([p[:, 64:], q[:, 64:]], axis=1))


def _prep_body(x_ref, w_ref, b_ref, t_ref, y_ref, t0_ref, t1_ref):
    y = jnp.maximum(x_ref[...] @ w_ref[...] + b_ref[...], 0.0)
    y_ref[...] = y
    t0_ref[...], t1_ref[...] = _tables(y, t_ref[0, 0])


def _tc_prep(x, wfc, bfc, t):
    row = pl.BlockSpec((RB, 128), lambda i: (i, 0))
    return pl.pallas_call(
        _prep_body,
        grid=(NRB,),
        in_specs=[
            row,
            pl.BlockSpec((128, 128), lambda i: (0, 0)),
            pl.BlockSpec((1, 128), lambda i: (0, 0)),
            pl.BlockSpec((1, 1), lambda i: (0, 0)),
        ],
        out_specs=[row, row, row],
        out_shape=[jax.ShapeDtypeStruct((N, 128), jnp.float32)] * 3,
    )(x, wfc, bfc.reshape(1, 128), t.reshape(1, 1))


def _layer_body(l, has_next, o0_ref, o1_ref, x_ref, w1_ref, b1_ref, g1_ref,
                e1_ref, w2_ref, b2_ref, gn_ref, bn_ref, t_ref, *out_refs):
    nx_ref = out_refs[0]
    o0 = o0_ref[...]
    o1 = o1_ref[...]
    xb = x_ref[...]
    S = jnp.concatenate([o0[:, :64], o1[:, :64]], axis=1)
    W = jnp.concatenate([o0[:, 64:], o1[:, 64:]], axis=1)
    o = W / (S + 1e-16) + xb
    h = o @ w1_ref[...] + b1_ref[...]
    h = jnp.maximum(_ln_in(h, g1_ref[...], e1_ref[...]), 0.0)
    h2 = h @ w2_ref[...] + b2_ref[...]
    if l == 0:
        nx = h2
    else:
        nx = xb + jnp.maximum(_ln_in(h2, gn_ref[...], bn_ref[...]), 0.0)
    nx_ref[...] = nx
    if has_next:
        t0, t1 = _tables(jnp.maximum(nx, 0.0), t_ref[0, 0])
        out_refs[1][...] = t0
        out_refs[2][...] = t1


def _tc_layer(l, out0, out1, x, p):
    has_next = l < 2
    row = pl.BlockSpec((RB, 128), lambda i: (i, 0))
    c1 = pl.BlockSpec((1, 256), lambda i: (0, 0))
    cw1 = pl.BlockSpec((128, 256), lambda i: (0, 0))
    cw2 = pl.BlockSpec((256, 128), lambda i: (0, 0))
    c2 = pl.BlockSpec((1, 128), lambda i: (0, 0))
    ct = pl.BlockSpec((1, 1), lambda i: (0, 0))
    gn = p['gn_%d' % l] if l else p['bm2_0']
    bn = p['bn_%d' % l] if l else p['bm2_0']
    tn = p['t%d' % (l + 1)] if has_next else p['t0']
    n_out = 3 if has_next else 1
    res = pl.pallas_call(
        functools.partial(_layer_body, l, has_next),
        grid=(NRB,),
        in_specs=[row, row, row, cw1, c1, c1, c1, cw2, c2, c2, c2, ct],
        out_specs=[row] * n_out,
        out_shape=[jax.ShapeDtypeStruct((N, 128), jnp.float32)] * n_out,
    )(out0[:N], out1[:N], x,
      p['Wm1_%d' % l], p['bm1_%d' % l].reshape(1, 256),
      p['g1_%d' % l].reshape(1, 256), p['be1_%d' % l].reshape(1, 256),
      p['Wm2_%d' % l], p['bm2_%d' % l].reshape(1, 128),
      gn.reshape(1, 128), bn.reshape(1, 128), tn.reshape(1, 1))
    return res if has_next else (res[0], None, None)


def _attn_body(x_ref, wphi_ref, bphi_ref, wa_ref, ba_ref, wb_ref, bb_ref,
               wc_ref, bc_ref, wr_ref, br_ref, wcls_ref, bcls_ref, out_ref):
    xb = x_ref[0]
    hp = jnp.maximum(xb @ wphi_ref[...] + bphi_ref[...], 0.0)
    a = jnp.tanh(hp @ wa_ref[...] + ba_ref[...])
    g = jax.nn.sigmoid(hp @ wb_ref[...] + bb_ref[...])
    s = ((a * g) @ wc_ref[...] + bc_ref[...])[:, :1]
    rowid = jax.lax.broadcasted_iota(jnp.int32, (512, 1), 0)
    s = jnp.where(rowid < 500, s, -1e30)
    s = s - jnp.max(s, axis=0, keepdims=True)
    e = jnp.exp(s)
    A = e / jnp.sum(e, axis=0, keepdims=True)
    h2 = jnp.sum(A * hp, axis=0, keepdims=True)
    bag = jnp.maximum(h2 @ wr_ref[...] + br_ref[...], 0.0)
    out_ref[0] = bag @ wcls_ref[...] + bcls_ref[...]


def _tc_attn(xp, p):
    D = 4 * H
    cw = pl.BlockSpec((D, D), lambda i: (0, 0))
    cb = pl.BlockSpec((1, D), lambda i: (0, 0))
    wc_pad = jnp.pad(p['Wc'], ((0, 0), (0, 127)))
    bc_pad = jnp.pad(p['bc'], ((0, 127))).reshape(1, 128)
    wcls_pad = jnp.pad(p['Wcls'], ((0, 0), (0, 126)))
    bcls_pad = jnp.pad(p['bcls'], ((0, 126))).reshape(1, 128)
    out = pl.pallas_call(
        _attn_body,
        grid=(B,),
        in_specs=[
            pl.BlockSpec((1, 512, D), lambda i: (i, 0, 0)),
            cw, cb, cw, cb, cw, cb,
            pl.BlockSpec((D, 128), lambda i: (0, 0)),
            pl.BlockSpec((1, 128), lambda i: (0, 0)),
            cw, cb,
            pl.BlockSpec((D, 128), lambda i: (0, 0)),
            pl.BlockSpec((1, 128), lambda i: (0, 0)),
        ],
        out_specs=pl.BlockSpec((1, 1, 128), lambda i: (i, 0, 0)),
        out_shape=jax.ShapeDtypeStruct((B, 1, 128), jnp.float32),
    )(xp, p['Wphi'], p['bphi'].reshape(1, D), p['Wa'], p['ba'].reshape(1, D),
      p['Wb'], p['bb'].reshape(1, D), wc_pad, bc_pad,
      p['Wrho'], p['brho'].reshape(1, D), wcls_pad, bcls_pad)
    return out.reshape(B, 128)[:, :2]


def kernel(x, edge_index, edge_latent, y, params):
    p = params
    src = edge_index[0].astype(jnp.int32)
    dst = edge_index[1].astype(jnp.int32)
    pad = E_PAD - E
    src_pad = jnp.concatenate([src, jnp.zeros((pad,), jnp.int32)])
    dst_pad = jnp.concatenate(
        [dst, N + (jnp.arange(pad, dtype=jnp.int32) % (N_PAD - N))])
    zrows = jnp.zeros((N_PAD, 128), jnp.float32)

    x, tab0, tab1 = _tc_prep(x, p['Wfc'], p['bfc'], p['t0'])
    cols = [x]
    for l in range(3):
        s0, s1 = _sc_segsum(tab0, tab1, src_pad, dst_pad, zrows)
        x, tab0, tab1 = _tc_layer(l, s0, s1, x, p)
        cols.append(x)
    x_ = jnp.concatenate(cols, axis=-1)
    xp = jnp.pad(x_.reshape(B, 500, 4 * H), ((0, 0), (0, 12), (0, 0)))
    return _tc_attn(xp, p)
